# Initial kernel scaffold; baseline (speedup 1.0000x reference)
import jax
import jax.numpy as jnp
from jax.experimental import pallas as pl


def kernel(input, emb_table, conv_w, conv_b, W_proj, b_proj, W_gate, b_gate):
    raise NotImplementedError("write your pallas kernel here")


# speedup vs baseline: 9.9008x; 9.9008x over previous
import jax
import jax.numpy as jnp
from jax.experimental import pallas as pl
from jax.experimental.pallas import tpu as pltpu

VOCAB = 96
ECHAR = 50
EWORD = 64
KSIZE = 5


def _fused_kernel(nb, mw):
    npos = mw - KSIZE + 1
    npairs = (npos - 1) // 2

    def body(idx_ref, emb_ref, wflat_ref, wpg_ref, cb2_ref, cb1_ref, bpg_ref,
             out_ref, tcat_ref, tpair_ref):
        @pl.when(pl.program_id(0) == 0)
        def _build_tables():
            emb = emb_ref[...]
            pieces = [emb @ wflat_ref[k * ECHAR:(k + 1) * ECHAR, :]
                      for k in range(KSIZE)]
            tcat = jnp.concatenate(pieces, axis=0)
            tcat_ref[...] = tcat
            z = jnp.zeros((VOCAB, EWORD), jnp.float32)
            left = jnp.concatenate([tcat, z], axis=0)
            right = jnp.concatenate([z, tcat], axis=0)
            tpair_ref[...] = jnp.concatenate([left, right], axis=1)

        idx = idx_ref[...]
        iota = jax.lax.broadcasted_iota(jnp.int32, (nb, VOCAB), 1)
        oh = jnp.concatenate(
            [(idx[:, j][:, None] == iota).astype(jnp.float32)
             for j in range(mw)], axis=1)

        tpair = tpair_ref[...]
        cb2 = cb2_ref[...]
        m = jnp.zeros((nb, EWORD), jnp.float32)
        for p in range(npairs):
            base = 2 * p * VOCAB
            a = jax.lax.dot_general(
                oh[:, base:base + 6 * VOCAB], tpair,
                (((1,), (0,)), ((), ())),
                preferred_element_type=jnp.float32) + cb2
            r = jnp.maximum(a, 0.0)
            m = jnp.maximum(m, jnp.maximum(r[:, :EWORD], r[:, EWORD:]))
        base = (npos - 1) * VOCAB
        a = jax.lax.dot_general(
            oh[:, base:base + KSIZE * VOCAB], tcat_ref[...],
            (((1,), (0,)), ((), ())),
            preferred_element_type=jnp.float32) + cb1_ref[...]
        m = jnp.maximum(m, jnp.maximum(a, 0.0))

        h = jax.lax.dot_general(
            m, wpg_ref[...], (((1,), (0,)), ((), ())),
            preferred_element_type=jnp.float32) + bpg_ref[...]
        proj = jnp.maximum(h[:, :EWORD], 0.0)
        gate = jax.nn.sigmoid(h[:, EWORD:])
        out_ref[...] = gate * proj + (1.0 - gate) * m

    return body


def kernel(input, emb_table, conv_w, conv_b, W_proj, b_proj, W_gate, b_gate):
    sl, bs, mw = input.shape
    n = sl * bs
    idx = input.reshape(n, mw)

    wflat = conv_w.transpose(2, 1, 0).reshape(KSIZE * ECHAR, EWORD)
    wpg = jnp.concatenate([W_proj.T, W_gate.T], axis=1)
    cb2 = jnp.concatenate([conv_b, conv_b])[None, :]
    cb1 = conv_b[None, :]
    bpg = jnp.concatenate([b_proj, b_gate])[None, :]

    nb = 512 if n % 512 == 0 else n
    grid = (n // nb,)

    out = pl.pallas_call(
        _fused_kernel(nb, mw),
        grid=grid,
        in_specs=[
            pl.BlockSpec((nb, mw), lambda i: (i, 0)),
            pl.BlockSpec((VOCAB, ECHAR), lambda i: (0, 0)),
            pl.BlockSpec((KSIZE * ECHAR, EWORD), lambda i: (0, 0)),
            pl.BlockSpec((EWORD, 2 * EWORD), lambda i: (0, 0)),
            pl.BlockSpec((1, 2 * EWORD), lambda i: (0, 0)),
            pl.BlockSpec((1, EWORD), lambda i: (0, 0)),
            pl.BlockSpec((1, 2 * EWORD), lambda i: (0, 0)),
        ],
        out_specs=pl.BlockSpec((nb, EWORD), lambda i: (i, 0)),
        out_shape=jax.ShapeDtypeStruct((n, EWORD), jnp.float32),
        scratch_shapes=[
            pltpu.VMEM((KSIZE * VOCAB, EWORD), jnp.float32),
            pltpu.VMEM(((KSIZE + 1) * VOCAB, 2 * EWORD), jnp.float32),
        ],
    )(idx, emb_table, wflat, wpg, cb2, cb1, bpg)
    return out.reshape(sl, bs, EWORD)


# speedup vs baseline: 12.4292x; 1.2554x over previous
import jax
import jax.numpy as jnp
from jax.experimental import pallas as pl
from jax.experimental.pallas import tpu as pltpu

VOCAB = 96
ECHAR = 50
EWORD = 64
KSIZE = 5


def _fused_kernel(nb, mw):
    npos = mw - KSIZE + 1
    nquads = npos // 4

    def body(idx_ref, emb_ref, wflat_ref, wpg_ref, cb4_ref, cb1_ref, bpg_ref,
             out_ref, tcat_ref, tquad_ref):
        @pl.when(pl.program_id(0) == 0)
        def _build_tables():
            emb = emb_ref[...]
            pieces = [emb @ wflat_ref[k * ECHAR:(k + 1) * ECHAR, :]
                      for k in range(KSIZE)]
            tcat = jnp.concatenate(pieces, axis=0)
            tcat_ref[...] = tcat.astype(jnp.bfloat16)
            z = jnp.zeros((VOCAB, EWORD), jnp.float32)
            cols = []
            for q in range(4):
                col = jnp.concatenate([z] * q + [tcat] + [z] * (3 - q), axis=0)
                cols.append(col)
            tquad_ref[...] = jnp.concatenate(cols, axis=1).astype(jnp.bfloat16)

        idx = idx_ref[...]
        iota = jax.lax.broadcasted_iota(jnp.int32, (nb, VOCAB), 1)
        oh = jnp.concatenate(
            [(idx[:, j][:, None] == iota).astype(jnp.bfloat16)
             for j in range(mw)], axis=1)

        tquad = tquad_ref[...]
        cb4 = cb4_ref[...]
        m = jnp.zeros((nb, EWORD), jnp.float32)
        for q in range(nquads):
            base = 4 * q * VOCAB
            a = jax.lax.dot_general(
                oh[:, base:base + 8 * VOCAB], tquad,
                (((1,), (0,)), ((), ())),
                preferred_element_type=jnp.float32) + cb4
            r = jnp.maximum(a, 0.0)
            m = jnp.maximum(
                m,
                jnp.maximum(
                    jnp.maximum(r[:, :EWORD], r[:, EWORD:2 * EWORD]),
                    jnp.maximum(r[:, 2 * EWORD:3 * EWORD], r[:, 3 * EWORD:])))
        for t in range(4 * nquads, npos):
            base = t * VOCAB
            a = jax.lax.dot_general(
                oh[:, base:base + KSIZE * VOCAB], tcat_ref[...],
                (((1,), (0,)), ((), ())),
                preferred_element_type=jnp.float32) + cb1_ref[...]
            m = jnp.maximum(m, jnp.maximum(a, 0.0))

        h = jax.lax.dot_general(
            m, wpg_ref[...], (((1,), (0,)), ((), ())),
            preferred_element_type=jnp.float32) + bpg_ref[...]
        proj = jnp.maximum(h[:, :EWORD], 0.0)
        gate = jax.nn.sigmoid(h[:, EWORD:])
        out_ref[...] = gate * proj + (1.0 - gate) * m

    return body


def kernel(input, emb_table, conv_w, conv_b, W_proj, b_proj, W_gate, b_gate):
    sl, bs, mw = input.shape
    n = sl * bs
    idx = input.reshape(n, mw)

    wflat = conv_w.transpose(2, 1, 0).reshape(KSIZE * ECHAR, EWORD)
    wpg = jnp.concatenate([W_proj.T, W_gate.T], axis=1)
    cb4 = jnp.concatenate([conv_b] * 4)[None, :]
    cb1 = conv_b[None, :]
    bpg = jnp.concatenate([b_proj, b_gate])[None, :]

    nb = 512 if n % 512 == 0 else n
    grid = (n // nb,)

    out = pl.pallas_call(
        _fused_kernel(nb, mw),
        grid=grid,
        in_specs=[
            pl.BlockSpec((nb, mw), lambda i: (i, 0)),
            pl.BlockSpec((VOCAB, ECHAR), lambda i: (0, 0)),
            pl.BlockSpec((KSIZE * ECHAR, EWORD), lambda i: (0, 0)),
            pl.BlockSpec((EWORD, 2 * EWORD), lambda i: (0, 0)),
            pl.BlockSpec((1, 4 * EWORD), lambda i: (0, 0)),
            pl.BlockSpec((1, EWORD), lambda i: (0, 0)),
            pl.BlockSpec((1, 2 * EWORD), lambda i: (0, 0)),
        ],
        out_specs=pl.BlockSpec((nb, EWORD), lambda i: (i, 0)),
        out_shape=jax.ShapeDtypeStruct((n, EWORD), jnp.float32),
        scratch_shapes=[
            pltpu.VMEM((KSIZE * VOCAB, EWORD), jnp.bfloat16),
            pltpu.VMEM((8 * VOCAB, 4 * EWORD), jnp.bfloat16),
        ],
    )(idx, emb_table, wflat, wpg, cb4, cb1, bpg)
    return out.reshape(sl, bs, EWORD)


# speedup vs baseline: 14.5370x; 1.1696x over previous
import jax
import jax.numpy as jnp
from jax.experimental import pallas as pl
from jax.experimental.pallas import tpu as pltpu

VOCAB = 96
VPAD = 128
ECHAR = 50
EWORD = 64
KSIZE = 5


def _fused_kernel(nb, mw):
    npos = mw - KSIZE + 1
    nquads = npos // 4

    def body(idx_ref, emb_ref, wflat_ref, wpg_ref, cb4_ref, cb1_ref, bpg_ref,
             out_ref, tcat_ref, tquad_ref):
        @pl.when(pl.program_id(0) == 0)
        def _build_tables():
            emb = emb_ref[...]
            z32 = jnp.zeros((VPAD - VOCAB, EWORD), jnp.float32)
            pieces = []
            for k in range(KSIZE):
                tk = emb @ wflat_ref[k * ECHAR:(k + 1) * ECHAR, :]
                pieces.append(tk)
                pieces.append(z32)
            tcat = jnp.concatenate(pieces, axis=0)
            tcat_ref[...] = tcat.astype(jnp.bfloat16)
            z128 = jnp.zeros((VPAD, EWORD), jnp.float32)
            cols = []
            for q in range(4):
                col = jnp.concatenate([z128] * q + [tcat] + [z128] * (3 - q),
                                      axis=0)
                cols.append(col)
            tquad_ref[...] = jnp.concatenate(cols, axis=1).astype(jnp.bfloat16)

        idx = idx_ref[...]
        iota = jax.lax.broadcasted_iota(jnp.int32, (nb, VPAD), 1)
        oh = jnp.concatenate(
            [(idx[:, j][:, None] == iota).astype(jnp.bfloat16)
             for j in range(mw)], axis=1)

        tquad = tquad_ref[...]
        cb4 = cb4_ref[...]
        m = jnp.zeros((nb, EWORD), jnp.float32)
        for q in range(nquads):
            base = 4 * q * VPAD
            a = jax.lax.dot_general(
                oh[:, base:base + 8 * VPAD], tquad,
                (((1,), (0,)), ((), ())),
                preferred_element_type=jnp.float32) + cb4
            r = jnp.maximum(a, 0.0)
            m = jnp.maximum(
                m,
                jnp.maximum(
                    jnp.maximum(r[:, :EWORD], r[:, EWORD:2 * EWORD]),
                    jnp.maximum(r[:, 2 * EWORD:3 * EWORD], r[:, 3 * EWORD:])))
        for t in range(4 * nquads, npos):
            base = t * VPAD
            a = jax.lax.dot_general(
                oh[:, base:base + KSIZE * VPAD], tcat_ref[...],
                (((1,), (0,)), ((), ())),
                preferred_element_type=jnp.float32) + cb1_ref[...]
            m = jnp.maximum(m, jnp.maximum(a, 0.0))

        h = jax.lax.dot_general(
            m, wpg_ref[...], (((1,), (0,)), ((), ())),
            preferred_element_type=jnp.float32) + bpg_ref[...]
        proj = jnp.maximum(h[:, :EWORD], 0.0)
        gate = jax.nn.sigmoid(h[:, EWORD:])
        out_ref[...] = gate * proj + (1.0 - gate) * m

    return body


def kernel(input, emb_table, conv_w, conv_b, W_proj, b_proj, W_gate, b_gate):
    sl, bs, mw = input.shape
    n = sl * bs
    idx = input.reshape(n, mw)

    wflat = conv_w.transpose(2, 1, 0).reshape(KSIZE * ECHAR, EWORD)
    wpg = jnp.concatenate([W_proj.T, W_gate.T], axis=1)
    cb4 = jnp.concatenate([conv_b] * 4)[None, :]
    cb1 = conv_b[None, :]
    bpg = jnp.concatenate([b_proj, b_gate])[None, :]

    nb = 512 if n % 512 == 0 else n
    grid = (n // nb,)

    out = pl.pallas_call(
        _fused_kernel(nb, mw),
        grid=grid,
        in_specs=[
            pl.BlockSpec((nb, mw), lambda i: (i, 0)),
            pl.BlockSpec((VOCAB, ECHAR), lambda i: (0, 0)),
            pl.BlockSpec((KSIZE * ECHAR, EWORD), lambda i: (0, 0)),
            pl.BlockSpec((EWORD, 2 * EWORD), lambda i: (0, 0)),
            pl.BlockSpec((1, 4 * EWORD), lambda i: (0, 0)),
            pl.BlockSpec((1, EWORD), lambda i: (0, 0)),
            pl.BlockSpec((1, 2 * EWORD), lambda i: (0, 0)),
        ],
        out_specs=pl.BlockSpec((nb, EWORD), lambda i: (i, 0)),
        out_shape=jax.ShapeDtypeStruct((n, EWORD), jnp.float32),
        scratch_shapes=[
            pltpu.VMEM((KSIZE * VPAD, EWORD), jnp.bfloat16),
            pltpu.VMEM((8 * VPAD, 4 * EWORD), jnp.bfloat16),
        ],
    )(idx, emb_table, wflat, wpg, cb4, cb1, bpg)
    return out.reshape(sl, bs, EWORD)


# speedup vs baseline: 16.8892x; 1.1618x over previous
import jax
import jax.numpy as jnp
from jax.experimental import pallas as pl
from jax.experimental.pallas import tpu as pltpu

VOCAB = 96
VPAD = 128
ECHAR = 50
EWORD = 64
KSIZE = 5


def _fused_kernel(nb, mw):
    npos = mw - KSIZE + 1
    nquads = npos // 4

    def body(idx_ref, emb_ref, wflat_ref, wpg_ref, cb1_ref, bpg_ref,
             out_ref, tcat_ref, tquad_ref):
        @pl.when(pl.program_id(0) == 0)
        def _build_tables():
            emb = emb_ref[...]
            z32 = jnp.zeros((VPAD - VOCAB, EWORD), jnp.float32)
            pieces = []
            for k in range(KSIZE):
                tk = emb @ wflat_ref[k * ECHAR:(k + 1) * ECHAR, :]
                pieces.append(tk)
                pieces.append(z32)
            tcat = jnp.concatenate(pieces, axis=0)
            tcat_ref[...] = tcat.astype(jnp.bfloat16)
            z128 = jnp.zeros((VPAD, EWORD), jnp.float32)
            cols = []
            for q in range(4):
                col = jnp.concatenate([z128] * q + [tcat] + [z128] * (3 - q),
                                      axis=0)
                cols.append(col)
            tquad_ref[...] = jnp.concatenate(cols, axis=1).astype(jnp.bfloat16)

        idx = idx_ref[...]
        iota = jax.lax.broadcasted_iota(jnp.int32, (nb, VPAD), 1)
        iotab = iota.astype(jnp.bfloat16)
        one = jnp.ones((nb, VPAD), jnp.bfloat16)
        zero = jnp.zeros((nb, VPAD), jnp.bfloat16)
        oh = jnp.concatenate(
            [jnp.where(idx[:, j][:, None] == iotab, one, zero)
             for j in range(mw)], axis=1)

        tquad = tquad_ref[...]
        m256 = None
        for q in range(nquads):
            base = 4 * q * VPAD
            a = jax.lax.dot_general(
                oh[:, base:base + 8 * VPAD], tquad,
                (((1,), (0,)), ((), ())),
                preferred_element_type=jnp.float32)
            m256 = a if m256 is None else jnp.maximum(m256, a)
        m = jnp.maximum(
            jnp.maximum(m256[:, :EWORD], m256[:, EWORD:2 * EWORD]),
            jnp.maximum(m256[:, 2 * EWORD:3 * EWORD], m256[:, 3 * EWORD:]))
        for t in range(4 * nquads, npos):
            base = t * VPAD
            a = jax.lax.dot_general(
                oh[:, base:base + KSIZE * VPAD], tcat_ref[...],
                (((1,), (0,)), ((), ())),
                preferred_element_type=jnp.float32)
            m = jnp.maximum(m, a)
        m = jnp.maximum(m + cb1_ref[...], 0.0)

        h = jax.lax.dot_general(
            m, wpg_ref[...], (((1,), (0,)), ((), ())),
            preferred_element_type=jnp.float32) + bpg_ref[...]
        proj = jnp.maximum(h[:, :EWORD], 0.0)
        gate = jax.nn.sigmoid(h[:, EWORD:])
        out_ref[...] = gate * proj + (1.0 - gate) * m

    return body


def kernel(input, emb_table, conv_w, conv_b, W_proj, b_proj, W_gate, b_gate):
    sl, bs, mw = input.shape
    n = sl * bs
    idx = input.reshape(n, mw).astype(jnp.bfloat16)

    wflat = conv_w.transpose(2, 1, 0).reshape(KSIZE * ECHAR, EWORD)
    wpg = jnp.concatenate([W_proj.T, W_gate.T], axis=1)
    cb1 = conv_b[None, :]
    bpg = jnp.concatenate([b_proj, b_gate])[None, :]

    nb = 512 if n % 512 == 0 else n
    grid = (n // nb,)

    out = pl.pallas_call(
        _fused_kernel(nb, mw),
        grid=grid,
        in_specs=[
            pl.BlockSpec((nb, mw), lambda i: (i, 0)),
            pl.BlockSpec((VOCAB, ECHAR), lambda i: (0, 0)),
            pl.BlockSpec((KSIZE * ECHAR, EWORD), lambda i: (0, 0)),
            pl.BlockSpec((EWORD, 2 * EWORD), lambda i: (0, 0)),
            pl.BlockSpec((1, EWORD), lambda i: (0, 0)),
            pl.BlockSpec((1, 2 * EWORD), lambda i: (0, 0)),
        ],
        out_specs=pl.BlockSpec((nb, EWORD), lambda i: (i, 0)),
        out_shape=jax.ShapeDtypeStruct((n, EWORD), jnp.float32),
        scratch_shapes=[
            pltpu.VMEM((KSIZE * VPAD, EWORD), jnp.bfloat16),
            pltpu.VMEM((8 * VPAD, 4 * EWORD), jnp.bfloat16),
        ],
    )(idx, emb_table, wflat, wpg, cb1, bpg)
    return out.reshape(sl, bs, EWORD)


# speedup vs baseline: 18.5224x; 1.0967x over previous
import jax
import jax.numpy as jnp
from jax.experimental import pallas as pl
from jax.experimental.pallas import tpu as pltpu

VOCAB = 96
VPAD = 128
ECHAR = 50
EWORD = 64
KSIZE = 5


def _fused_kernel(nb, mw):
    npos = mw - KSIZE + 1
    nquads = npos // 4

    def body(idx_ref, emb_ref, wflat_ref, wpg_ref, cb1_ref, bpg_ref,
             out_ref, tcat_ref, tquad_ref):
        @pl.when(pl.program_id(0) == 0)
        def _build_tables():
            emb = emb_ref[...]
            z32 = jnp.zeros((VPAD - VOCAB, EWORD), jnp.float32)
            pieces = []
            for k in range(KSIZE):
                tk = emb @ wflat_ref[k * ECHAR:(k + 1) * ECHAR, :]
                pieces.append(tk)
                pieces.append(z32)
            tcat = jnp.concatenate(pieces, axis=0)
            tcat_ref[...] = tcat.astype(jnp.bfloat16)
            z128 = jnp.zeros((VPAD, EWORD), jnp.float32)
            cols = []
            for q in range(4):
                col = jnp.concatenate([z128] * q + [tcat] + [z128] * (3 - q),
                                      axis=0)
                cols.append(col)
            tquad_ref[...] = jnp.concatenate(cols, axis=1).astype(jnp.bfloat16)

        idx = idx_ref[...]
        iota = jax.lax.broadcasted_iota(jnp.int32, (nb, VPAD), 1)
        iotab = iota.astype(jnp.bfloat16)
        one = jnp.ones((nb, VPAD), jnp.bfloat16)
        zero = jnp.zeros((nb, VPAD), jnp.bfloat16)
        oh = jnp.concatenate(
            [jnp.where(idx[:, j][:, None] == iotab, one, zero)
             for j in range(mw)], axis=1)

        tquad = tquad_ref[...]
        m256 = None
        for q in range(nquads):
            base = 4 * q * VPAD
            a = jax.lax.dot_general(
                oh[:, base:base + 8 * VPAD], tquad,
                (((1,), (0,)), ((), ())),
                preferred_element_type=jnp.float32)
            m256 = a if m256 is None else jnp.maximum(m256, a)
        m = jnp.maximum(
            jnp.maximum(m256[:, :EWORD], m256[:, EWORD:2 * EWORD]),
            jnp.maximum(m256[:, 2 * EWORD:3 * EWORD], m256[:, 3 * EWORD:]))
        for t in range(4 * nquads, npos):
            base = t * VPAD
            a = jax.lax.dot_general(
                oh[:, base:base + KSIZE * VPAD], tcat_ref[...],
                (((1,), (0,)), ((), ())),
                preferred_element_type=jnp.float32)
            m = jnp.maximum(m, a)
        m = jnp.maximum(m + cb1_ref[...], 0.0)

        h = jax.lax.dot_general(
            m, wpg_ref[...], (((1,), (0,)), ((), ())),
            preferred_element_type=jnp.float32) + bpg_ref[...]
        proj = jnp.maximum(h[:, :EWORD], 0.0)
        gate = jax.nn.sigmoid(h[:, EWORD:])
        out_ref[...] = gate * proj + (1.0 - gate) * m

    return body


def kernel(input, emb_table, conv_w, conv_b, W_proj, b_proj, W_gate, b_gate):
    sl, bs, mw = input.shape
    n = sl * bs
    idx = input.reshape(n, mw).astype(jnp.bfloat16)

    wflat = conv_w.transpose(2, 1, 0).reshape(KSIZE * ECHAR, EWORD)
    wpg = jnp.concatenate([W_proj.T, W_gate.T], axis=1)
    cb1 = conv_b[None, :]
    bpg = jnp.concatenate([b_proj, b_gate])[None, :]

    nb = 1024 if n % 1024 == 0 else n
    grid = (n // nb,)

    out = pl.pallas_call(
        _fused_kernel(nb, mw),
        grid=grid,
        in_specs=[
            pl.BlockSpec((nb, mw), lambda i: (i, 0)),
            pl.BlockSpec((VOCAB, ECHAR), lambda i: (0, 0)),
            pl.BlockSpec((KSIZE * ECHAR, EWORD), lambda i: (0, 0)),
            pl.BlockSpec((EWORD, 2 * EWORD), lambda i: (0, 0)),
            pl.BlockSpec((1, EWORD), lambda i: (0, 0)),
            pl.BlockSpec((1, 2 * EWORD), lambda i: (0, 0)),
        ],
        out_specs=pl.BlockSpec((nb, EWORD), lambda i: (i, 0)),
        out_shape=jax.ShapeDtypeStruct((n, EWORD), jnp.float32),
        scratch_shapes=[
            pltpu.VMEM((KSIZE * VPAD, EWORD), jnp.bfloat16),
            pltpu.VMEM((8 * VPAD, 4 * EWORD), jnp.bfloat16),
        ],
    )(idx, emb_table, wflat, wpg, cb1, bpg)
    return out.reshape(sl, bs, EWORD)


# speedup vs baseline: 19.0379x; 1.0278x over previous
import jax
import jax.numpy as jnp
from jax.experimental import pallas as pl
from jax.experimental.pallas import tpu as pltpu

VOCAB = 96
VPAD = 128
ECHAR = 50
EWORD = 64
KSIZE = 5


def _fused_kernel(nb, mw):
    npos = mw - KSIZE + 1
    nquads = npos // 4

    def body(idx_ref, emb_ref, wflat_ref, wpg_ref, cb1_ref, bpg_ref,
             out_ref, tcat_ref, tquad_ref):
        @pl.when(pl.program_id(0) == 0)
        def _build_tables():
            emb = emb_ref[...]
            z32 = jnp.zeros((VPAD - VOCAB, EWORD), jnp.float32)
            pieces = []
            for k in range(KSIZE):
                tk = emb @ wflat_ref[k * ECHAR:(k + 1) * ECHAR, :]
                pieces.append(tk)
                pieces.append(z32)
            tcat = jnp.concatenate(pieces, axis=0)
            tcat_ref[...] = tcat.astype(jnp.bfloat16)
            z128 = jnp.zeros((VPAD, EWORD), jnp.float32)
            cols = []
            for q in range(4):
                col = jnp.concatenate([z128] * q + [tcat] + [z128] * (3 - q),
                                      axis=0)
                cols.append(col)
            tquad_ref[...] = jnp.concatenate(cols, axis=1).astype(jnp.bfloat16)

        idx = idx_ref[...]
        iota = jax.lax.broadcasted_iota(jnp.int32, (nb, VPAD), 1)
        iotab = iota.astype(jnp.bfloat16)
        one = jnp.ones((nb, VPAD), jnp.bfloat16)
        zero = jnp.zeros((nb, VPAD), jnp.bfloat16)
        oh = jnp.concatenate(
            [jnp.where(idx[:, j][:, None] == iotab, one, zero)
             for j in range(mw)], axis=1)

        tquad = tquad_ref[...]
        m256 = None
        for q in range(nquads):
            base = 4 * q * VPAD
            a = jax.lax.dot_general(
                oh[:, base:base + 8 * VPAD], tquad,
                (((1,), (0,)), ((), ())),
                preferred_element_type=jnp.float32)
            m256 = a if m256 is None else jnp.maximum(m256, a)
        m = jnp.maximum(
            jnp.maximum(m256[:, :EWORD], m256[:, EWORD:2 * EWORD]),
            jnp.maximum(m256[:, 2 * EWORD:3 * EWORD], m256[:, 3 * EWORD:]))
        for t in range(4 * nquads, npos):
            base = t * VPAD
            a = jax.lax.dot_general(
                oh[:, base:base + KSIZE * VPAD], tcat_ref[...],
                (((1,), (0,)), ((), ())),
                preferred_element_type=jnp.float32)
            m = jnp.maximum(m, a)
        m = jnp.maximum(m + cb1_ref[...], 0.0)

        h = jax.lax.dot_general(
            m, wpg_ref[...], (((1,), (0,)), ((), ())),
            preferred_element_type=jnp.float32) + bpg_ref[...]
        proj = jnp.maximum(h[:, :EWORD], 0.0)
        gate = jax.nn.sigmoid(h[:, EWORD:])
        out_ref[...] = gate * proj + (1.0 - gate) * m

    return body


def kernel(input, emb_table, conv_w, conv_b, W_proj, b_proj, W_gate, b_gate):
    sl, bs, mw = input.shape
    n = sl * bs
    idx = input.reshape(n, mw).astype(jnp.bfloat16)

    wflat = conv_w.transpose(2, 1, 0).reshape(KSIZE * ECHAR, EWORD)
    wpg = jnp.concatenate([W_proj.T, W_gate.T], axis=1)
    cb1 = conv_b[None, :]
    bpg = jnp.concatenate([b_proj, b_gate])[None, :]

    nb = 2048 if n % 2048 == 0 else n
    grid = (n // nb,)

    out = pl.pallas_call(
        _fused_kernel(nb, mw),
        grid=grid,
        in_specs=[
            pl.BlockSpec((nb, mw), lambda i: (i, 0)),
            pl.BlockSpec((VOCAB, ECHAR), lambda i: (0, 0)),
            pl.BlockSpec((KSIZE * ECHAR, EWORD), lambda i: (0, 0)),
            pl.BlockSpec((EWORD, 2 * EWORD), lambda i: (0, 0)),
            pl.BlockSpec((1, EWORD), lambda i: (0, 0)),
            pl.BlockSpec((1, 2 * EWORD), lambda i: (0, 0)),
        ],
        out_specs=pl.BlockSpec((nb, EWORD), lambda i: (i, 0)),
        out_shape=jax.ShapeDtypeStruct((n, EWORD), jnp.float32),
        scratch_shapes=[
            pltpu.VMEM((KSIZE * VPAD, EWORD), jnp.bfloat16),
            pltpu.VMEM((8 * VPAD, 4 * EWORD), jnp.bfloat16),
        ],
    )(idx, emb_table, wflat, wpg, cb1, bpg)
    return out.reshape(sl, bs, EWORD)
